# Initial kernel scaffold; baseline (speedup 1.0000x reference)
import jax
import jax.numpy as jnp
from jax.experimental import pallas as pl


def kernel(x, Wq, bq, Wk, bk, Wv, bv, Wp, bp, lepe_w, lepe_b):
    raise NotImplementedError("write your pallas kernel here")


# speedup vs baseline: 1.2197x; 1.2197x over previous
import functools

import jax
import jax.numpy as jnp
from jax import lax
from jax.experimental import pallas as pl
from jax.experimental.pallas import tpu as pltpu
from jax.experimental.pallas import tpu_sc as plsc

C = 192
GS = 8
P = GS * GS
N = 2304
K = 3
SCALE = C ** (-0.5)
ROWB = 64
WB = 16
NW = 32
NB = N // NW
ROWF = P * C


def _pool_body(xw_ref, dsx_ref):
    dsx_ref[...] = jnp.max(xw_ref[...], axis=1)


def _pool(xw):
    return pl.pallas_call(
        _pool_body,
        grid=(N // ROWB,),
        in_specs=[pl.BlockSpec((ROWB, P, C), lambda i: (i, 0, 0))],
        out_specs=pl.BlockSpec((ROWB, C), lambda i: (i, 0)),
        out_shape=jax.ShapeDtypeStruct((N, C), jnp.float32),
    )(xw)


def _topk_body(dsx_blk_ref, dsx_all_ref, w_ref, idx_ref):
    s = lax.dot_general(
        dsx_blk_ref[...], dsx_all_ref[...],
        (((1,), (1,)), ((), ())),
        preferred_element_type=jnp.float32)
    iota = lax.broadcasted_iota(jnp.int32, s.shape, 1)
    scores, idxs = [], []
    for _ in range(K):
        m = jnp.max(s, axis=1, keepdims=True)
        sel = jnp.min(jnp.where(s == m, iota, N), axis=1, keepdims=True)
        scores.append(m)
        idxs.append(sel)
        s = jnp.where(iota == sel, -jnp.inf, s)
    sc = jnp.concatenate(scores, axis=1)
    e = jnp.exp(sc - sc[:, :1])
    w_ref[...] = e / jnp.sum(e, axis=1, keepdims=True)
    idx_ref[...] = jnp.concatenate(idxs, axis=1)


def _topk(dsx):
    return pl.pallas_call(
        _topk_body,
        grid=(N // ROWB,),
        in_specs=[
            pl.BlockSpec((ROWB, C), lambda i: (i, 0)),
            pl.BlockSpec((N, C), lambda i: (0, 0)),
        ],
        out_specs=[
            pl.BlockSpec((ROWB, K), lambda i: (i, 0)),
            pl.BlockSpec((ROWB, K), lambda i: (i, 0)),
        ],
        out_shape=[
            jax.ShapeDtypeStruct((N, K), jnp.float32),
            jax.ShapeDtypeStruct((N, K), jnp.int32),
        ],
    )(dsx, dsx)


def _sc_ctx_body(xw_hbm, idxp_hbm, w3_hbm, ctx_hbm, idx1_v, w3_v, rows_v,
                 out_v, sem):
    wid = lax.axis_index("s") * 2 + lax.axis_index("c")
    base = wid * NB

    def win_body(i, carry):
        pltpu.sync_copy(idxp_hbm.at[wid, i], idx1_v)
        pltpu.sync_copy(w3_hbm.at[base + i], w3_v)
        pltpu.async_copy(xw_hbm.at[idx1_v], rows_v, sem).wait()
        w0 = w3_v[0, :]
        w1 = w3_v[1, :]
        w2 = w3_v[2, :]

        def chunk(j, c2):
            sl = pl.ds(j * 16, 16)
            out_v[0, sl] = (rows_v[0, sl] * w0 + rows_v[1, sl] * w1
                            + rows_v[2, sl] * w2)
            return c2

        lax.fori_loop(0, ROWF // 16, chunk, 0, unroll=8)
        pltpu.sync_copy(out_v, ctx_hbm.at[pl.ds(base + i, 1)])
        return carry

    lax.fori_loop(0, NB, win_body, 0)


def _sc_ctx(xw2, idxp, w3):
    mesh = plsc.VectorSubcoreMesh(core_axis_name="c", subcore_axis_name="s")
    fn = functools.partial(
        pl.kernel,
        mesh=mesh,
        out_type=jax.ShapeDtypeStruct((N, ROWF), jnp.float32),
        scratch_types=[
            pltpu.VMEM((8,), jnp.int32),
            pltpu.VMEM((K, 16), jnp.float32),
            pltpu.VMEM((8, ROWF), jnp.float32),
            pltpu.VMEM((1, ROWF), jnp.float32),
            pltpu.SemaphoreType.DMA,
        ],
    )(_sc_ctx_body)
    return fn(xw2, idxp, w3)


def _attn_body(xw_ref, ctx_ref, wqt, bq, wkt, bk, wvt, bv, wpt, bp, wl, lb,
               out_ref):
    x3 = xw_ref[...]
    x2 = x3.reshape(WB * P, C)
    c2 = ctx_ref[...].reshape(WB * P, C)
    q = jnp.dot(x2, wqt[...], preferred_element_type=jnp.float32) + bq[...]
    kk = jnp.dot(c2, wkt[...], preferred_element_type=jnp.float32) + bk[...]
    v = jnp.dot(c2, wvt[...], preferred_element_type=jnp.float32) + bv[...]

    pad = jnp.zeros((WB, 16, C), jnp.float32)
    xp = jnp.concatenate([pad, x3, pad], axis=1)
    s_id = lax.broadcasted_iota(jnp.int32, (1, P, 1), 1) % GS
    wlv = wl[...]
    lepe = jnp.zeros((WB, P, C), jnp.float32) + lb[...]
    for dy in range(3):
        for dx in range(3):
            o = (dy - 1) * GS + (dx - 1)
            sl = lax.slice_in_dim(xp, 16 + o, 16 + o + P, axis=1)
            cm = ((s_id + (dx - 1) >= 0) & (s_id + (dx - 1) < GS))
            term = wlv[3 * dy + dx][None, None, :] * sl
            lepe = lepe + jnp.where(cm, term, 0.0)

    q3 = q.reshape(WB, P, C)
    k3 = kk.reshape(WB, P, C)
    v3 = v.reshape(WB, P, C)
    outs = []
    for w in range(WB):
        a = lax.dot_general(q3[w], k3[w], (((1,), (1,)), ((), ())),
                            preferred_element_type=jnp.float32) * SCALE
        a = a - jnp.max(a, axis=1, keepdims=True)
        e = jnp.exp(a)
        p = e / jnp.sum(e, axis=1, keepdims=True)
        outs.append(jnp.dot(p, v3[w], preferred_element_type=jnp.float32))
    ao = jnp.stack(outs, axis=0) + lepe
    res = jnp.dot(ao.reshape(WB * P, C), wpt[...],
                  preferred_element_type=jnp.float32) + bp[...]
    out_ref[...] = res.reshape(WB, P, C)


def _attn(xw, ctx, WqT, bq, WkT, bk, WvT, bv, WpT, bp, wl, lb):
    wspec = pl.BlockSpec((C, C), lambda i: (0, 0))
    bspec = pl.BlockSpec((1, C), lambda i: (0, 0))
    return pl.pallas_call(
        _attn_body,
        grid=(N // WB,),
        in_specs=[
            pl.BlockSpec((WB, P, C), lambda i: (i, 0, 0)),
            pl.BlockSpec((WB, P, C), lambda i: (i, 0, 0)),
            wspec, bspec, wspec, bspec, wspec, bspec, wspec, bspec,
            pl.BlockSpec((9, C), lambda i: (0, 0)),
            pl.BlockSpec((1, 1, C), lambda i: (0, 0, 0)),
        ],
        out_specs=pl.BlockSpec((WB, P, C), lambda i: (i, 0, 0)),
        out_shape=jax.ShapeDtypeStruct((N, P, C), jnp.float32),
    )(xw, ctx, WqT, bq, WkT, bk, WvT, bv, WpT, bp, wl, lb)


def kernel(x, Wq, bq, Wk, bk, Wv, bv, Wp, bp, lepe_w, lepe_b):
    B = x.shape[0]
    gh = x.shape[2] // GS
    gw = x.shape[3] // GS
    xw = jnp.transpose(x, (0, 2, 3, 1)).reshape(B, gh, GS, gw, GS, C)
    xw = jnp.transpose(xw, (0, 1, 3, 2, 4, 5)).reshape(N, P, C)

    dsx = _pool(xw)
    wts, idx = _topk(dsx)

    idxp = jnp.concatenate(
        [idx, jnp.broadcast_to(idx[:, :1], (N, 5))], axis=1)
    idxp = idxp.reshape(NW, NB, 8)
    wb16 = jnp.broadcast_to(wts.reshape(N, K, 1), (N, K, 16))
    ctx2 = _sc_ctx(xw.reshape(N, ROWF), idxp, wb16)
    ctx = ctx2.reshape(N, P, C)

    wl = jnp.transpose(lepe_w.reshape(C, 9), (1, 0))
    out_w = _attn(xw, ctx, Wq.T, bq[None], Wk.T, bk[None], Wv.T, bv[None],
                  Wp.T, bp[None], wl, lepe_b[None, None])

    out = out_w.reshape(B, gh, gw, GS, GS, C)
    out = jnp.transpose(out, (0, 5, 1, 3, 2, 4)).reshape(
        B, C, gh * GS, gw * GS)
    return out


# speedup vs baseline: 1.7350x; 1.4226x over previous
import functools

import jax
import jax.numpy as jnp
from jax import lax
from jax.experimental import pallas as pl
from jax.experimental.pallas import tpu as pltpu
from jax.experimental.pallas import tpu_sc as plsc

C = 192
GS = 8
P = GS * GS
N = 2304
K = 3
SCALE = C ** (-0.5)
ROWB = 64
WB = 16
NW = 32
NB = N // NW
ROWF = P * C


def _pool_body(xw_ref, dsx_ref):
    dsx_ref[...] = jnp.max(xw_ref[...], axis=1)


def _pool(xw):
    return pl.pallas_call(
        _pool_body,
        grid=(N // ROWB,),
        in_specs=[pl.BlockSpec((ROWB, P, C), lambda i: (i, 0, 0))],
        out_specs=pl.BlockSpec((ROWB, C), lambda i: (i, 0)),
        out_shape=jax.ShapeDtypeStruct((N, C), jnp.float32),
    )(xw)


def _topk_body(dsx_blk_ref, dsx_all_ref, w_ref, idx_ref):
    s = lax.dot_general(
        dsx_blk_ref[...], dsx_all_ref[...],
        (((1,), (1,)), ((), ())),
        preferred_element_type=jnp.float32)
    iota = lax.broadcasted_iota(jnp.int32, s.shape, 1)
    scores, idxs = [], []
    for _ in range(K):
        m = jnp.max(s, axis=1, keepdims=True)
        sel = jnp.min(jnp.where(s == m, iota, N), axis=1, keepdims=True)
        scores.append(m)
        idxs.append(sel)
        s = jnp.where(iota == sel, -jnp.inf, s)
    sc = jnp.concatenate(scores, axis=1)
    e = jnp.exp(sc - sc[:, :1])
    w_ref[...] = e / jnp.sum(e, axis=1, keepdims=True)
    idx_ref[...] = jnp.concatenate(idxs, axis=1)


def _topk(dsx):
    return pl.pallas_call(
        _topk_body,
        grid=(N // ROWB,),
        in_specs=[
            pl.BlockSpec((ROWB, C), lambda i: (i, 0)),
            pl.BlockSpec((N, C), lambda i: (0, 0)),
        ],
        out_specs=[
            pl.BlockSpec((ROWB, K), lambda i: (i, 0)),
            pl.BlockSpec((ROWB, K), lambda i: (i, 0)),
        ],
        out_shape=[
            jax.ShapeDtypeStruct((N, K), jnp.float32),
            jax.ShapeDtypeStruct((N, K), jnp.int32),
        ],
    )(dsx, dsx)


def _sc_ctx_body(xw_hbm, idxp_hbm, w3_hbm, ctx_hbm, ix_v, wv, rows_v,
                 out_v, sem):
    wid = lax.axis_index("s") * 2 + lax.axis_index("c")
    base = wid * NB

    def win_body(i, carry):
        pltpu.sync_copy(idxp_hbm.at[wid, i, pl.ds(0, 4)], ix_v)
        pltpu.sync_copy(w3_hbm.at[base + i], wv)
        pltpu.async_copy(xw_hbm.at[ix_v], rows_v, sem).wait()
        w0 = wv[0, :]
        w1 = wv[1, :]
        w2 = wv[2, :]

        def chunk(j, c2):
            sl = pl.ds(j * 16, 16)
            out_v[0, sl] = (rows_v[0, sl] * w0 + rows_v[1, sl] * w1
                            + rows_v[2, sl] * w2)
            return c2

        lax.fori_loop(0, ROWF // 16, chunk, 0, unroll=8)
        pltpu.sync_copy(out_v, ctx_hbm.at[pl.ds(base + i, 1)])
        return carry

    lax.fori_loop(0, NB, win_body, 0)


def _sc_ctx(xw2, idxp, w3):
    mesh = plsc.VectorSubcoreMesh(core_axis_name="c", subcore_axis_name="s")
    fn = functools.partial(
        pl.kernel,
        mesh=mesh,
        out_type=jax.ShapeDtypeStruct((N, ROWF), jnp.float32),
        scratch_types=[
            pltpu.VMEM((4,), jnp.int32),
            pltpu.VMEM((K, 16), jnp.float32),
            pltpu.VMEM((4, ROWF), jnp.float32),
            pltpu.VMEM((1, ROWF), jnp.float32),
            pltpu.SemaphoreType.DMA,
        ],
    )(_sc_ctx_body)
    return fn(xw2, idxp, w3)


def _attn_body(xw_ref, ctx_ref, wqt, bq, wkt, bk, wvt, bv, wpt, bp, wl, lb,
               out_ref):
    x3 = xw_ref[...]
    x2 = x3.reshape(WB * P, C)
    c2 = ctx_ref[...].reshape(WB * P, C)
    q = jnp.dot(x2, wqt[...], preferred_element_type=jnp.float32) + bq[...]
    kk = jnp.dot(c2, wkt[...], preferred_element_type=jnp.float32) + bk[...]
    v = jnp.dot(c2, wvt[...], preferred_element_type=jnp.float32) + bv[...]

    pad = jnp.zeros((WB, 16, C), jnp.float32)
    xp = jnp.concatenate([pad, x3, pad], axis=1)
    s_id = lax.broadcasted_iota(jnp.int32, (1, P, 1), 1) % GS
    wlv = wl[...]
    lepe = jnp.zeros((WB, P, C), jnp.float32) + lb[...]
    for dy in range(3):
        for dx in range(3):
            o = (dy - 1) * GS + (dx - 1)
            sl = lax.slice_in_dim(xp, 16 + o, 16 + o + P, axis=1)
            cm = ((s_id + (dx - 1) >= 0) & (s_id + (dx - 1) < GS))
            term = wlv[3 * dy + dx][None, None, :] * sl
            lepe = lepe + jnp.where(cm, term, 0.0)

    q3 = q.reshape(WB, P, C)
    k3 = kk.reshape(WB, P, C)
    v3 = v.reshape(WB, P, C)
    outs = []
    for w in range(WB):
        a = lax.dot_general(q3[w], k3[w], (((1,), (1,)), ((), ())),
                            preferred_element_type=jnp.float32) * SCALE
        a = a - jnp.max(a, axis=1, keepdims=True)
        e = jnp.exp(a)
        p = e / jnp.sum(e, axis=1, keepdims=True)
        outs.append(jnp.dot(p, v3[w], preferred_element_type=jnp.float32))
    ao = jnp.stack(outs, axis=0) + lepe
    res = jnp.dot(ao.reshape(WB * P, C), wpt[...],
                  preferred_element_type=jnp.float32) + bp[...]
    out_ref[...] = res.reshape(WB, P, C)


def _attn(xw, ctx, WqT, bq, WkT, bk, WvT, bv, WpT, bp, wl, lb):
    wspec = pl.BlockSpec((C, C), lambda i: (0, 0))
    bspec = pl.BlockSpec((1, C), lambda i: (0, 0))
    return pl.pallas_call(
        _attn_body,
        grid=(N // WB,),
        in_specs=[
            pl.BlockSpec((WB, P, C), lambda i: (i, 0, 0)),
            pl.BlockSpec((WB, P, C), lambda i: (i, 0, 0)),
            wspec, bspec, wspec, bspec, wspec, bspec, wspec, bspec,
            pl.BlockSpec((9, C), lambda i: (0, 0)),
            pl.BlockSpec((1, 1, C), lambda i: (0, 0, 0)),
        ],
        out_specs=pl.BlockSpec((WB, P, C), lambda i: (i, 0, 0)),
        out_shape=jax.ShapeDtypeStruct((N, P, C), jnp.float32),
    )(xw, ctx, WqT, bq, WkT, bk, WvT, bv, WpT, bp, wl, lb)


def kernel(x, Wq, bq, Wk, bk, Wv, bv, Wp, bp, lepe_w, lepe_b):
    B = x.shape[0]
    gh = x.shape[2] // GS
    gw = x.shape[3] // GS
    xw = jnp.transpose(x, (0, 2, 3, 1)).reshape(B, gh, GS, gw, GS, C)
    xw = jnp.transpose(xw, (0, 1, 3, 2, 4, 5)).reshape(N, P, C)

    dsx = _pool(xw)
    wts, idx = _topk(dsx)

    idxp = jnp.concatenate(
        [idx, jnp.broadcast_to(idx[:, :1], (N, 5))], axis=1).reshape(NW, NB, 8)
    wb16 = jnp.broadcast_to(wts.reshape(N, K, 1), (N, K, 16))
    ctx2 = _sc_ctx(xw.reshape(N, ROWF), idxp, wb16)
    ctx = ctx2.reshape(N, P, C)

    wl = jnp.transpose(lepe_w.reshape(C, 9), (1, 0))
    out_w = _attn(xw, ctx, Wq.T, bq[None], Wk.T, bk[None], Wv.T, bv[None],
                  Wp.T, bp[None], wl, lepe_b[None, None])

    out = out_w.reshape(B, gh, gw, GS, GS, C)
    out = jnp.transpose(out, (0, 5, 1, 3, 2, 4)).reshape(
        B, C, gh * GS, gw * GS)
    return out


# speedup vs baseline: 1.7887x; 1.0309x over previous
import functools

import jax
import jax.numpy as jnp
from jax import lax
from jax.experimental import pallas as pl
from jax.experimental.pallas import tpu as pltpu
from jax.experimental.pallas import tpu_sc as plsc

C = 192
GS = 8
P = GS * GS
N = 2304
K = 3
SCALE = C ** (-0.5)
ROWB = 64
WB = 16
NW = 32
NB = N // NW
ROWF = P * C


def _pool_body(xw_ref, dsx_ref):
    dsx_ref[...] = jnp.max(xw_ref[...], axis=1)


def _pool(xw):
    return pl.pallas_call(
        _pool_body,
        grid=(N // ROWB,),
        in_specs=[pl.BlockSpec((ROWB, P, C), lambda i: (i, 0, 0))],
        out_specs=pl.BlockSpec((ROWB, C), lambda i: (i, 0)),
        out_shape=jax.ShapeDtypeStruct((N, C), jnp.float32),
    )(xw)


def _topk_body(dsx_blk_ref, dsx_all_ref, w_ref, idx_ref):
    s = lax.dot_general(
        dsx_blk_ref[...], dsx_all_ref[...],
        (((1,), (1,)), ((), ())),
        preferred_element_type=jnp.float32)
    iota = lax.broadcasted_iota(jnp.int32, s.shape, 1)
    scores, idxs = [], []
    for _ in range(K):
        m = jnp.max(s, axis=1, keepdims=True)
        sel = jnp.min(jnp.where(s == m, iota, N), axis=1, keepdims=True)
        scores.append(m)
        idxs.append(sel)
        s = jnp.where(iota == sel, -jnp.inf, s)
    sc = jnp.concatenate(scores, axis=1)
    e = jnp.exp(sc - sc[:, :1])
    w_ref[...] = e / jnp.sum(e, axis=1, keepdims=True)
    idx_ref[...] = jnp.concatenate(idxs, axis=1)


def _topk(dsx):
    return pl.pallas_call(
        _topk_body,
        grid=(N // ROWB,),
        in_specs=[
            pl.BlockSpec((ROWB, C), lambda i: (i, 0)),
            pl.BlockSpec((N, C), lambda i: (0, 0)),
        ],
        out_specs=[
            pl.BlockSpec((ROWB, K), lambda i: (i, 0)),
            pl.BlockSpec((ROWB, K), lambda i: (i, 0)),
        ],
        out_shape=[
            jax.ShapeDtypeStruct((N, K), jnp.float32),
            jax.ShapeDtypeStruct((N, K), jnp.int32),
        ],
    )(dsx, dsx)


def _sc_ctx_body(xw_hbm, idxp_hbm, w3_hbm, ctx_hbm, ix_a, ix_b, wv,
                 rows_a, rows_b, out_v,
                 fsem_a, fsem_b, gsem_a, gsem_b, osem):
    wid = lax.axis_index("s") * 2 + lax.axis_index("c")
    base = wid * NB
    pltpu.sync_copy(w3_hbm.at[wid], wv)

    def fetch_start(i, ix, fsem):
        pltpu.async_copy(idxp_hbm.at[wid, i, pl.ds(0, 4)], ix, fsem)

    def fetch_wait(ix, fsem):
        pltpu.make_async_copy(idxp_hbm.at[0, 0, pl.ds(0, 4)], ix,
                              fsem).wait()

    def gather_start(ix, rows, gsem):
        pltpu.async_copy(xw_hbm.at[ix], rows, gsem)

    def gather_wait(rows, gsem):
        pltpu.make_async_copy(xw_hbm.at[pl.ds(0, 4)], rows, gsem).wait()

    def combine(i, rows):
        w0 = wv[pl.ds(K * 16 * i, 16)]
        w1 = wv[pl.ds(K * 16 * i + 16, 16)]
        w2 = wv[pl.ds(K * 16 * i + 32, 16)]

        def chunk(j, c2):
            sl = pl.ds(j * 16, 16)
            out_v[sl] = (rows[0, sl] * w0 + rows[1, sl] * w1
                         + rows[2, sl] * w2)
            return c2

        lax.fori_loop(0, ROWF // 16, chunk, 0, unroll=8)

    def out_start(i):
        pltpu.async_copy(out_v, ctx_hbm.at[base + i], osem)

    def out_wait():
        pltpu.make_async_copy(out_v, ctx_hbm.at[0], osem).wait()

    pltpu.sync_copy(idxp_hbm.at[wid, 0, pl.ds(0, 4)], ix_a)
    gather_start(ix_a, rows_a, gsem_a)
    fetch_start(1, ix_b, fsem_b)
    ng = NB // 2

    def body(g, carry):
        ia = 2 * g
        ib = 2 * g + 1
        fetch_wait(ix_b, fsem_b)
        gather_start(ix_b, rows_b, gsem_b)
        gather_wait(rows_a, gsem_a)

        @pl.when(g > 0)
        def _():
            out_wait()

        combine(ia, rows_a)
        out_start(ia)

        @pl.when(g < ng - 1)
        def _():
            fetch_start(ia + 2, ix_a, fsem_a)
            fetch_wait(ix_a, fsem_a)
            gather_start(ix_a, rows_a, gsem_a)

        gather_wait(rows_b, gsem_b)
        out_wait()
        combine(ib, rows_b)
        out_start(ib)

        @pl.when(g < ng - 1)
        def _():
            fetch_start(ib + 2, ix_b, fsem_b)

        return carry

    lax.fori_loop(0, ng, body, 0)
    out_wait()


def _sc_ctx(xw2, idxp, w3):
    mesh = plsc.VectorSubcoreMesh(core_axis_name="c", subcore_axis_name="s")
    fn = functools.partial(
        pl.kernel,
        mesh=mesh,
        out_type=jax.ShapeDtypeStruct((N, ROWF), jnp.float32),
        scratch_types=[
            pltpu.VMEM((4,), jnp.int32),
            pltpu.VMEM((4,), jnp.int32),
            pltpu.VMEM((NB * K * 16,), jnp.float32),
            pltpu.VMEM((4, ROWF), jnp.float32),
            pltpu.VMEM((4, ROWF), jnp.float32),
            pltpu.VMEM((ROWF,), jnp.float32),
            pltpu.SemaphoreType.DMA,
            pltpu.SemaphoreType.DMA,
            pltpu.SemaphoreType.DMA,
            pltpu.SemaphoreType.DMA,
            pltpu.SemaphoreType.DMA,
        ],
    )(_sc_ctx_body)
    return fn(xw2, idxp, w3)


def _attn_body(xw_ref, ctx_ref, wqt, bq, wkt, bk, wvt, bv, wpt, bp, wl, lb,
               out_ref):
    x3 = xw_ref[...]
    x2 = x3.reshape(WB * P, C)
    c2 = ctx_ref[...].reshape(WB * P, C)
    q = jnp.dot(x2, wqt[...], preferred_element_type=jnp.float32) + bq[...]
    kk = jnp.dot(c2, wkt[...], preferred_element_type=jnp.float32) + bk[...]
    v = jnp.dot(c2, wvt[...], preferred_element_type=jnp.float32) + bv[...]

    pad = jnp.zeros((WB, 16, C), jnp.float32)
    xp = jnp.concatenate([pad, x3, pad], axis=1)
    s_id = lax.broadcasted_iota(jnp.int32, (1, P, 1), 1) % GS
    wlv = wl[...]
    lepe = jnp.zeros((WB, P, C), jnp.float32) + lb[...]
    for dy in range(3):
        for dx in range(3):
            o = (dy - 1) * GS + (dx - 1)
            sl = lax.slice_in_dim(xp, 16 + o, 16 + o + P, axis=1)
            cm = ((s_id + (dx - 1) >= 0) & (s_id + (dx - 1) < GS))
            term = wlv[3 * dy + dx][None, None, :] * sl
            lepe = lepe + jnp.where(cm, term, 0.0)

    q3 = q.reshape(WB, P, C)
    k3 = kk.reshape(WB, P, C)
    v3 = v.reshape(WB, P, C)
    outs = []
    for w in range(WB):
        a = lax.dot_general(q3[w], k3[w], (((1,), (1,)), ((), ())),
                            preferred_element_type=jnp.float32) * SCALE
        a = a - jnp.max(a, axis=1, keepdims=True)
        e = jnp.exp(a)
        p = e / jnp.sum(e, axis=1, keepdims=True)
        outs.append(jnp.dot(p, v3[w], preferred_element_type=jnp.float32))
    ao = jnp.stack(outs, axis=0) + lepe
    res = jnp.dot(ao.reshape(WB * P, C), wpt[...],
                  preferred_element_type=jnp.float32) + bp[...]
    out_ref[...] = res.reshape(WB, P, C)


def _attn(xw, ctx, WqT, bq, WkT, bk, WvT, bv, WpT, bp, wl, lb):
    wspec = pl.BlockSpec((C, C), lambda i: (0, 0))
    bspec = pl.BlockSpec((1, C), lambda i: (0, 0))
    return pl.pallas_call(
        _attn_body,
        grid=(N // WB,),
        in_specs=[
            pl.BlockSpec((WB, P, C), lambda i: (i, 0, 0)),
            pl.BlockSpec((WB, P, C), lambda i: (i, 0, 0)),
            wspec, bspec, wspec, bspec, wspec, bspec, wspec, bspec,
            pl.BlockSpec((9, C), lambda i: (0, 0)),
            pl.BlockSpec((1, 1, C), lambda i: (0, 0, 0)),
        ],
        out_specs=pl.BlockSpec((WB, P, C), lambda i: (i, 0, 0)),
        out_shape=jax.ShapeDtypeStruct((N, P, C), jnp.float32),
    )(xw, ctx, WqT, bq, WkT, bk, WvT, bv, WpT, bp, wl, lb)


def kernel(x, Wq, bq, Wk, bk, Wv, bv, Wp, bp, lepe_w, lepe_b):
    B = x.shape[0]
    gh = x.shape[2] // GS
    gw = x.shape[3] // GS
    xw = jnp.transpose(x, (0, 2, 3, 1)).reshape(B, gh, GS, gw, GS, C)
    xw = jnp.transpose(xw, (0, 1, 3, 2, 4, 5)).reshape(N, P, C)

    dsx = _pool(xw)
    wts, idx = _topk(dsx)

    idxp = jnp.concatenate(
        [idx, jnp.broadcast_to(idx[:, :1], (N, 5))], axis=1).reshape(NW, NB, 8)
    wb16 = jnp.broadcast_to(
        wts.reshape(NW, NB * K, 1), (NW, NB * K, 16)).reshape(NW, NB * K * 16)
    ctx2 = _sc_ctx(xw.reshape(N, ROWF), idxp, wb16)
    ctx = ctx2.reshape(N, P, C)

    wl = jnp.transpose(lepe_w.reshape(C, 9), (1, 0))
    out_w = _attn(xw, ctx, Wq.T, bq[None], Wk.T, bk[None], Wv.T, bv[None],
                  Wp.T, bp[None], wl, lepe_b[None, None])

    out = out_w.reshape(B, gh, gw, GS, GS, C)
    out = jnp.transpose(out, (0, 5, 1, 3, 2, 4)).reshape(
        B, C, gh * GS, gw * GS)
    return out


# speedup vs baseline: 1.7894x; 1.0004x over previous
import functools

import jax
import jax.numpy as jnp
from jax import lax
from jax.experimental import pallas as pl
from jax.experimental.pallas import tpu as pltpu
from jax.experimental.pallas import tpu_sc as plsc

C = 192
GS = 8
P = GS * GS
N = 2304
K = 3
SCALE = C ** (-0.5)
ROWB = 64
WB = 16
NW = 32
NB = N // NW
ROWF = P * C


def _pool_body(xw_ref, dsx_ref):
    dsx_ref[...] = jnp.max(xw_ref[...], axis=1)


def _pool(xw):
    return pl.pallas_call(
        _pool_body,
        grid=(N // ROWB,),
        in_specs=[pl.BlockSpec((ROWB, P, C), lambda i: (i, 0, 0))],
        out_specs=pl.BlockSpec((ROWB, C), lambda i: (i, 0)),
        out_shape=jax.ShapeDtypeStruct((N, C), jnp.float32),
    )(xw)


def _topk_body(dsx_blk_ref, dsx_all_ref, w_ref, idx_ref):
    s = lax.dot_general(
        dsx_blk_ref[...], dsx_all_ref[...],
        (((1,), (1,)), ((), ())),
        preferred_element_type=jnp.float32)
    iota = lax.broadcasted_iota(jnp.int32, s.shape, 1)
    scores, idxs = [], []
    for _ in range(K):
        m = jnp.max(s, axis=1, keepdims=True)
        sel = jnp.min(jnp.where(s == m, iota, N), axis=1, keepdims=True)
        scores.append(m)
        idxs.append(sel)
        s = jnp.where(iota == sel, -jnp.inf, s)
    sc = jnp.concatenate(scores, axis=1)
    e = jnp.exp(sc - sc[:, :1])
    w_ref[...] = e / jnp.sum(e, axis=1, keepdims=True)
    idx_ref[...] = jnp.concatenate(idxs, axis=1)


def _topk(dsx):
    return pl.pallas_call(
        _topk_body,
        grid=(N // ROWB,),
        in_specs=[
            pl.BlockSpec((ROWB, C), lambda i: (i, 0)),
            pl.BlockSpec((N, C), lambda i: (0, 0)),
        ],
        out_specs=[
            pl.BlockSpec((ROWB, K), lambda i: (i, 0)),
            pl.BlockSpec((ROWB, K), lambda i: (i, 0)),
        ],
        out_shape=[
            jax.ShapeDtypeStruct((N, K), jnp.float32),
            jax.ShapeDtypeStruct((N, K), jnp.int32),
        ],
    )(dsx, dsx)


def _sc_ctx_body(xw_hbm, idxp_hbm, w3_hbm, ctx_hbm, ix_a, ix_b, wv,
                 rows_a, rows_b, out_a, out_b,
                 fsem_a, fsem_b, gsem_a, gsem_b, osem_a, osem_b):
    wid = lax.axis_index("s") * 2 + lax.axis_index("c")
    base = wid * NB
    pltpu.sync_copy(w3_hbm.at[wid], wv)

    def fetch_start(i, ix, fsem):
        pltpu.async_copy(idxp_hbm.at[wid, i, pl.ds(0, 4)], ix, fsem)

    def fetch_wait(ix, fsem):
        pltpu.make_async_copy(idxp_hbm.at[0, 0, pl.ds(0, 4)], ix,
                              fsem).wait()

    def gather_start(ix, rows, gsem):
        pltpu.async_copy(xw_hbm.at[ix], rows, gsem)

    def gather_wait(rows, gsem):
        pltpu.make_async_copy(xw_hbm.at[pl.ds(0, 4)], rows, gsem).wait()

    def combine(i, rows, out):
        w0 = wv[pl.ds(K * 16 * i, 16)]
        w1 = wv[pl.ds(K * 16 * i + 16, 16)]
        w2 = wv[pl.ds(K * 16 * i + 32, 16)]

        def chunk(j, c2):
            sl = pl.ds(j * 16, 16)
            out[sl] = (rows[0, sl] * w0 + rows[1, sl] * w1
                       + rows[2, sl] * w2)
            return c2

        lax.fori_loop(0, ROWF // 16, chunk, 0, unroll=8)

    def out_start(i, out, osem):
        pltpu.async_copy(out, ctx_hbm.at[base + i], osem)

    def out_wait(out, osem):
        pltpu.make_async_copy(out, ctx_hbm.at[0], osem).wait()

    pltpu.sync_copy(idxp_hbm.at[wid, 0, pl.ds(0, 4)], ix_a)
    gather_start(ix_a, rows_a, gsem_a)
    fetch_start(1, ix_b, fsem_b)
    ng = NB // 2

    def body(g, carry):
        ia = 2 * g
        ib = 2 * g + 1
        fetch_wait(ix_b, fsem_b)
        gather_start(ix_b, rows_b, gsem_b)
        gather_wait(rows_a, gsem_a)

        @pl.when(g < ng - 1)
        def _():
            fetch_start(ia + 2, ix_a, fsem_a)

        @pl.when(g > 0)
        def _():
            out_wait(out_a, osem_a)

        combine(ia, rows_a, out_a)
        out_start(ia, out_a, osem_a)

        @pl.when(g < ng - 1)
        def _():
            fetch_wait(ix_a, fsem_a)
            gather_start(ix_a, rows_a, gsem_a)

        gather_wait(rows_b, gsem_b)

        @pl.when(g > 0)
        def _():
            out_wait(out_b, osem_b)

        combine(ib, rows_b, out_b)
        out_start(ib, out_b, osem_b)

        @pl.when(g < ng - 1)
        def _():
            fetch_start(ib + 2, ix_b, fsem_b)

        return carry

    lax.fori_loop(0, ng, body, 0)
    out_wait(out_a, osem_a)
    out_wait(out_b, osem_b)


def _sc_ctx(xw2, idxp, w3):
    mesh = plsc.VectorSubcoreMesh(core_axis_name="c", subcore_axis_name="s")
    fn = functools.partial(
        pl.kernel,
        mesh=mesh,
        out_type=jax.ShapeDtypeStruct((N, ROWF), jnp.float32),
        scratch_types=[
            pltpu.VMEM((4,), jnp.int32),
            pltpu.VMEM((4,), jnp.int32),
            pltpu.VMEM((NB * K * 16,), jnp.float32),
            pltpu.VMEM((4, ROWF), jnp.float32),
            pltpu.VMEM((4, ROWF), jnp.float32),
            pltpu.VMEM((ROWF,), jnp.float32),
            pltpu.VMEM((ROWF,), jnp.float32),
            pltpu.SemaphoreType.DMA,
            pltpu.SemaphoreType.DMA,
            pltpu.SemaphoreType.DMA,
            pltpu.SemaphoreType.DMA,
            pltpu.SemaphoreType.DMA,
            pltpu.SemaphoreType.DMA,
        ],
    )(_sc_ctx_body)
    return fn(xw2, idxp, w3)


def _attn_body(xw_ref, ctx_ref, wqt, bq, wkt, bk, wvt, bv, wpt, bp, wl, lb,
               out_ref):
    x3 = xw_ref[...]
    x2 = x3.reshape(WB * P, C)
    c2 = ctx_ref[...].reshape(WB * P, C)
    q = jnp.dot(x2, wqt[...], preferred_element_type=jnp.float32) + bq[...]
    kk = jnp.dot(c2, wkt[...], preferred_element_type=jnp.float32) + bk[...]
    v = jnp.dot(c2, wvt[...], preferred_element_type=jnp.float32) + bv[...]

    pad = jnp.zeros((WB, 16, C), jnp.float32)
    xp = jnp.concatenate([pad, x3, pad], axis=1)
    s_id = lax.broadcasted_iota(jnp.int32, (1, P, 1), 1) % GS
    wlv = wl[...]
    lepe = jnp.zeros((WB, P, C), jnp.float32) + lb[...]
    for dy in range(3):
        for dx in range(3):
            o = (dy - 1) * GS + (dx - 1)
            sl = lax.slice_in_dim(xp, 16 + o, 16 + o + P, axis=1)
            cm = ((s_id + (dx - 1) >= 0) & (s_id + (dx - 1) < GS))
            term = wlv[3 * dy + dx][None, None, :] * sl
            lepe = lepe + jnp.where(cm, term, 0.0)

    q3 = q.reshape(WB, P, C)
    k3 = kk.reshape(WB, P, C)
    v3 = v.reshape(WB, P, C)
    outs = []
    for w in range(WB):
        a = lax.dot_general(q3[w], k3[w], (((1,), (1,)), ((), ())),
                            preferred_element_type=jnp.float32) * SCALE
        a = a - jnp.max(a, axis=1, keepdims=True)
        e = jnp.exp(a)
        p = e / jnp.sum(e, axis=1, keepdims=True)
        outs.append(jnp.dot(p, v3[w], preferred_element_type=jnp.float32))
    ao = jnp.stack(outs, axis=0) + lepe
    res = jnp.dot(ao.reshape(WB * P, C), wpt[...],
                  preferred_element_type=jnp.float32) + bp[...]
    out_ref[...] = res.reshape(WB, P, C)


def _attn(xw, ctx, WqT, bq, WkT, bk, WvT, bv, WpT, bp, wl, lb):
    wspec = pl.BlockSpec((C, C), lambda i: (0, 0))
    bspec = pl.BlockSpec((1, C), lambda i: (0, 0))
    return pl.pallas_call(
        _attn_body,
        grid=(N // WB,),
        in_specs=[
            pl.BlockSpec((WB, P, C), lambda i: (i, 0, 0)),
            pl.BlockSpec((WB, P, C), lambda i: (i, 0, 0)),
            wspec, bspec, wspec, bspec, wspec, bspec, wspec, bspec,
            pl.BlockSpec((9, C), lambda i: (0, 0)),
            pl.BlockSpec((1, 1, C), lambda i: (0, 0, 0)),
        ],
        out_specs=pl.BlockSpec((WB, P, C), lambda i: (i, 0, 0)),
        out_shape=jax.ShapeDtypeStruct((N, P, C), jnp.float32),
    )(xw, ctx, WqT, bq, WkT, bk, WvT, bv, WpT, bp, wl, lb)


def kernel(x, Wq, bq, Wk, bk, Wv, bv, Wp, bp, lepe_w, lepe_b):
    B = x.shape[0]
    gh = x.shape[2] // GS
    gw = x.shape[3] // GS
    xw = jnp.transpose(x, (0, 2, 3, 1)).reshape(B, gh, GS, gw, GS, C)
    xw = jnp.transpose(xw, (0, 1, 3, 2, 4, 5)).reshape(N, P, C)

    dsx = _pool(xw)
    wts, idx = _topk(dsx)

    idxp = jnp.concatenate(
        [idx, jnp.broadcast_to(idx[:, :1], (N, 5))], axis=1).reshape(NW, NB, 8)
    wb16 = jnp.broadcast_to(
        wts.reshape(NW, NB * K, 1), (NW, NB * K, 16)).reshape(NW, NB * K * 16)
    ctx2 = _sc_ctx(xw.reshape(N, ROWF), idxp, wb16)
    ctx = ctx2.reshape(N, P, C)

    wl = jnp.transpose(lepe_w.reshape(C, 9), (1, 0))
    out_w = _attn(xw, ctx, Wq.T, bq[None], Wk.T, bk[None], Wv.T, bv[None],
                  Wp.T, bp[None], wl, lepe_b[None, None])

    out = out_w.reshape(B, gh, gw, GS, GS, C)
    out = jnp.transpose(out, (0, 5, 1, 3, 2, 4)).reshape(
        B, C, gh * GS, gw * GS)
    return out
